# baseline (speedup 1.0000x reference)
import functools
import jax
import jax.numpy as jnp
from jax import lax
from jax.experimental import pallas as pl
from jax.experimental.pallas import tpu as pltpu
from jax.experimental.pallas import tpu_sc as plsc

N_NODES = 10000
N_PAD = 10240
N_EDGES = 320000
D = 128
NC = 2
NS = 16
NW = NC * NS
G = 128
GPT = 80
E_PAD = NW * GPT * G
NBUF = 2
ROWS_PER_TILE = N_PAD // NS

_MESH = plsc.VectorSubcoreMesh(core_axis_name="c", subcore_axis_name="s")


def _deg_body(dst_hbm, out_hbm, idx_v, ones_v, zvec_v, deg_sp, sem):
    c = lax.axis_index("c")
    s = lax.axis_index("s")
    wid = s * NC + c
    g0 = wid * GPT
    for j in range(G // 16):
        ones_v[pl.ds(j * 16, 16)] = jnp.ones((16,), jnp.float32)

    def zfill(i, carry):
        zvec_v[pl.ds(i * 16, 16)] = jnp.zeros((16,), jnp.float32)
        return carry

    lax.fori_loop(0, ROWS_PER_TILE // 16, zfill, 0)
    pltpu.sync_copy(dst_hbm.at[pl.ds(g0, GPT)], idx_v)
    pltpu.sync_copy(zvec_v, deg_sp.at[pl.ds(s * ROWS_PER_TILE, ROWS_PER_TILE)])
    plsc.subcore_barrier()

    def body(t, carry):
        for b in range(16):
            j = t * 16 + b
            pltpu.async_copy(ones_v, deg_sp.at[idx_v.at[j]], sem, add=True)
        for b in range(16):
            j = t * 16 + b
            pltpu.make_async_copy(ones_v, deg_sp.at[idx_v.at[j]], sem).wait()
        return carry

    lax.fori_loop(0, GPT // 16, body, 0)
    plsc.subcore_barrier()
    pltpu.sync_copy(deg_sp.at[pl.ds(s * ROWS_PER_TILE, ROWS_PER_TILE)],
                    out_hbm.at[c, pl.ds(s * ROWS_PER_TILE, ROWS_PER_TILE)])


_deg_call = functools.partial(
    pl.kernel,
    out_type=jax.ShapeDtypeStruct((NC, N_PAD), jnp.float32),
    mesh=_MESH,
    scratch_types=[
        pltpu.VMEM((GPT, G), jnp.int32),
        pltpu.VMEM((G,), jnp.float32),
        pltpu.VMEM((ROWS_PER_TILE,), jnp.float32),
        pltpu.VMEM_SHARED((N_PAD,), jnp.float32),
        pltpu.SemaphoreType.DMA,
    ],
)(_deg_body)


_HALF = GPT // 2


def _prop_body(y_hbm, src_hbm, dst_hbm, out_hbm,
               sidx_v, didx_v, msgs, gsems, acc_sp):
    c = lax.axis_index("c")
    s = lax.axis_index("s")
    wid = s * NC + c
    g0 = wid * GPT

    def zfill(i, carry):
        for k in range(D // 16):
            msgs[0][i, pl.ds(k * 16, 16)] = jnp.zeros((16,), jnp.float32)
        return carry

    lax.fori_loop(0, G, zfill, 0)
    for r in range(ROWS_PER_TILE // G):
        pltpu.sync_copy(msgs[0],
                        acc_sp.at[pl.ds(s * ROWS_PER_TILE + r * G, G)])
    plsc.subcore_barrier()

    def start_gather(j, b):
        pltpu.async_copy(y_hbm.at[sidx_v.at[j]], msgs[b], gsems[b])

    def wait_gather(j, b):
        pltpu.make_async_copy(y_hbm.at[sidx_v.at[j]], msgs[b], gsems[b]).wait()

    for h in range(2):
        pltpu.sync_copy(src_hbm.at[pl.ds(g0 + h * _HALF, _HALF)], sidx_v)
        pltpu.sync_copy(dst_hbm.at[pl.ds(g0 + h * _HALF, _HALF)], didx_v)
        for b in range(NBUF):
            start_gather(b, b)

        def body(t, carry):
            for b in range(NBUF):
                j = t * NBUF + b
                wait_gather(j, b)
                pltpu.sync_copy(msgs[b], acc_sp.at[didx_v.at[j]], add=True)

                @pl.when(j + NBUF < _HALF)
                def _():
                    start_gather(j + NBUF, b)
            return carry

        lax.fori_loop(0, _HALF // NBUF, body, 0)

    plsc.subcore_barrier()
    pltpu.sync_copy(acc_sp.at[pl.ds(s * ROWS_PER_TILE, ROWS_PER_TILE)],
                    out_hbm.at[c, pl.ds(s * ROWS_PER_TILE, ROWS_PER_TILE)])


_prop_call = functools.partial(
    pl.kernel,
    out_type=jax.ShapeDtypeStruct((NC, N_PAD, D), jnp.float32),
    mesh=_MESH,
    scratch_types=[
        pltpu.VMEM((_HALF, G), jnp.int32),
        pltpu.VMEM((_HALF, G), jnp.int32),
        [pltpu.VMEM((G, D), jnp.float32) for _ in range(NBUF)],
        [pltpu.SemaphoreType.DMA for _ in range(NBUF)],
        pltpu.VMEM_SHARED((N_PAD, D), jnp.float32),
    ],
)(_prop_body)


_BLK = 10240
_GRID = N_PAD // _BLK

_row_spec = pl.BlockSpec((_BLK, D), lambda i: (i, 0))
_col_spec = pl.BlockSpec((_BLK, 1), lambda i: (i, 0))
_w_spec = pl.BlockSpec((D, D), lambda i: (0, 0))
_b_spec = pl.BlockSpec((1, D), lambda i: (0, 0))


def _mm1_body(p0_ref, p1_ref, x_ref, w_ref, dinv_ref, y_ref):
    deg = p0_ref[...] + p1_ref[...] + 1.0
    dinv = lax.rsqrt(deg)
    dinv_ref[...] = dinv
    y_ref[...] = jnp.dot(x_ref[...], w_ref[...],
                         preferred_element_type=jnp.float32) * dinv


def _mm1_call(p0, p1, x, w):
    return pl.pallas_call(
        _mm1_body,
        grid=(_GRID,),
        in_specs=[_col_spec, _col_spec, _row_spec, _w_spec],
        out_specs=[_col_spec, _row_spec],
        out_shape=[
            jax.ShapeDtypeStruct((N_PAD, 1), jnp.float32),
            jax.ShapeDtypeStruct((N_PAD, D), jnp.float32),
        ],
    )(p0, p1, x, w)


def _mm2_body(a0_ref, a1_ref, y1_ref, dinv_ref, b_ref, w_ref, y2_ref):
    dinv = dinv_ref[...]
    h = (a0_ref[...] + a1_ref[...] + y1_ref[...]) * dinv + b_ref[...]
    h = jnp.maximum(h, 0.0)
    y2_ref[...] = jnp.dot(h, w_ref[...],
                          preferred_element_type=jnp.float32) * dinv


def _mm2_call(a0, a1, y1, dinv, b, w):
    return pl.pallas_call(
        _mm2_body,
        grid=(_GRID,),
        in_specs=[_row_spec, _row_spec, _row_spec, _col_spec, _b_spec, _w_spec],
        out_specs=_row_spec,
        out_shape=jax.ShapeDtypeStruct((N_PAD, D), jnp.float32),
    )(a0, a1, y1, dinv, b, w)


def _fin_body(a0_ref, a1_ref, y2_ref, dinv_ref, b_ref, out_ref):
    out_ref[...] = ((a0_ref[...] + a1_ref[...] + y2_ref[...]) * dinv_ref[...]
                    + b_ref[...])


_FBLK = 2000
_fin_row_spec = pl.BlockSpec((_FBLK, D), lambda i: (i, 0))
_fin_col_spec = pl.BlockSpec((_FBLK, 1), lambda i: (i, 0))


def _fin_call(a0, a1, y2, dinv, b):
    return pl.pallas_call(
        _fin_body,
        grid=(N_NODES // _FBLK,),
        in_specs=[_fin_row_spec, _fin_row_spec, _fin_row_spec,
                  _fin_col_spec, _b_spec],
        out_specs=_fin_row_spec,
        out_shape=jax.ShapeDtypeStruct((N_NODES, D), jnp.float32),
    )(a0, a1, y2, dinv, b)


@jax.jit
def _run(x, src2d, dst2d, W1, b1, W2, b2):
    b1r = b1.reshape(1, D)
    b2r = b2.reshape(1, D)

    degp = _deg_call(dst2d)
    p0 = degp[0].reshape(N_PAD, 1)
    p1 = degp[1].reshape(N_PAD, 1)
    dinv, y1 = _mm1_call(p0, p1, x, W1)

    acc1 = _prop_call(y1, src2d, dst2d)
    y2 = _mm2_call(acc1[0], acc1[1], y1, dinv, b1r, W2)

    acc2 = _prop_call(y2, src2d, dst2d)
    return _fin_call(acc2[0], acc2[1], y2, dinv, b2r)


def kernel(x, edge_index, W1, b1, W2, b2):
    src = edge_index[0].astype(jnp.int32)
    dst = edge_index[1].astype(jnp.int32)
    pad = N_NODES + (jnp.arange(E_PAD - N_EDGES, dtype=jnp.int32)
                     % (N_PAD - N_NODES))
    src2d = jnp.concatenate([src, pad]).reshape(-1, G)
    dst2d = jnp.concatenate([dst, pad]).reshape(-1, G)
    return _run(x, src2d, dst2d, W1, b1, W2, b2)


# speedup vs baseline: 1.0012x; 1.0012x over previous
import functools
import jax
import jax.numpy as jnp
from jax import lax
from jax.experimental import pallas as pl
from jax.experimental.pallas import tpu as pltpu
from jax.experimental.pallas import tpu_sc as plsc

N_NODES = 10000
N_PAD = 10240
N_EDGES = 320000
D = 128
NC = 2
NS = 16
NW = NC * NS
G = 128
GPT = 80
E_PAD = NW * GPT * G
NBUF = 2
ROWS_PER_TILE = N_PAD // NS

_MESH = plsc.VectorSubcoreMesh(core_axis_name="c", subcore_axis_name="s")


def _deg_body(dst_hbm, out_hbm, idx_v, ones_v, zvec_v, deg_sp, sem):
    c = lax.axis_index("c")
    s = lax.axis_index("s")
    wid = s * NC + c
    g0 = wid * GPT
    for j in range(G // 16):
        ones_v[pl.ds(j * 16, 16)] = jnp.ones((16,), jnp.float32)

    def zfill(i, carry):
        zvec_v[pl.ds(i * 16, 16)] = jnp.zeros((16,), jnp.float32)
        return carry

    lax.fori_loop(0, ROWS_PER_TILE // 16, zfill, 0)
    pltpu.sync_copy(dst_hbm.at[pl.ds(g0, GPT)], idx_v)
    pltpu.sync_copy(zvec_v, deg_sp.at[pl.ds(s * ROWS_PER_TILE, ROWS_PER_TILE)])
    plsc.subcore_barrier()

    def body(t, carry):
        for b in range(16):
            j = t * 16 + b
            pltpu.async_copy(ones_v, deg_sp.at[idx_v.at[j]], sem, add=True)
        for b in range(16):
            j = t * 16 + b
            pltpu.make_async_copy(ones_v, deg_sp.at[idx_v.at[j]], sem).wait()
        return carry

    lax.fori_loop(0, GPT // 16, body, 0)
    plsc.subcore_barrier()
    pltpu.sync_copy(deg_sp.at[pl.ds(s * ROWS_PER_TILE, ROWS_PER_TILE)],
                    out_hbm.at[c, pl.ds(s * ROWS_PER_TILE, ROWS_PER_TILE)])


_deg_call = functools.partial(
    pl.kernel,
    out_type=jax.ShapeDtypeStruct((NC, N_PAD), jnp.float32),
    mesh=_MESH,
    scratch_types=[
        pltpu.VMEM((GPT, G), jnp.int32),
        pltpu.VMEM((G,), jnp.float32),
        pltpu.VMEM((ROWS_PER_TILE,), jnp.float32),
        pltpu.VMEM_SHARED((N_PAD,), jnp.float32),
        pltpu.SemaphoreType.DMA,
    ],
)(_deg_body)


_HALF = GPT // 2


def _prop_body(y_hbm, src_hbm, dst_hbm, out_hbm,
               sidx_v, didx_v, msgs, gsems, acc_sp):
    c = lax.axis_index("c")
    s = lax.axis_index("s")
    wid = s * NC + c
    g0 = wid * GPT

    def zfill(i, carry):
        for k in range(D // 16):
            msgs[0][i, pl.ds(k * 16, 16)] = jnp.zeros((16,), jnp.float32)
        return carry

    lax.fori_loop(0, G, zfill, 0)
    for r in range(ROWS_PER_TILE // G):
        pltpu.sync_copy(msgs[0],
                        acc_sp.at[pl.ds(s * ROWS_PER_TILE + r * G, G)])
    plsc.subcore_barrier()

    def start_gather(j, b):
        pltpu.async_copy(y_hbm.at[sidx_v.at[j]], msgs[b], gsems[b])

    def wait_gather(j, b):
        pltpu.make_async_copy(y_hbm.at[sidx_v.at[j]], msgs[b], gsems[b]).wait()

    for h in range(2):
        pltpu.sync_copy(src_hbm.at[pl.ds(g0 + h * _HALF, _HALF)], sidx_v)
        pltpu.sync_copy(dst_hbm.at[pl.ds(g0 + h * _HALF, _HALF)], didx_v)
        for b in range(NBUF):
            start_gather(b, b)

        def body(t, carry):
            for b in range(NBUF):
                j = t * NBUF + b
                wait_gather(j, b)
                pltpu.sync_copy(msgs[b], acc_sp.at[didx_v.at[j]], add=True)

                @pl.when(j + NBUF < _HALF)
                def _():
                    start_gather(j + NBUF, b)
            return carry

        lax.fori_loop(0, _HALF // NBUF, body, 0)

    plsc.subcore_barrier()
    pltpu.sync_copy(acc_sp.at[pl.ds(s * ROWS_PER_TILE, ROWS_PER_TILE)],
                    out_hbm.at[c, pl.ds(s * ROWS_PER_TILE, ROWS_PER_TILE)])


_prop_call = functools.partial(
    pl.kernel,
    out_type=jax.ShapeDtypeStruct((NC, N_PAD, D), jnp.float32),
    mesh=_MESH,
    scratch_types=[
        pltpu.VMEM((_HALF, G), jnp.int32),
        pltpu.VMEM((_HALF, G), jnp.int32),
        [pltpu.VMEM((G, D), jnp.float32) for _ in range(NBUF)],
        [pltpu.SemaphoreType.DMA for _ in range(NBUF)],
        pltpu.VMEM_SHARED((N_PAD, D), jnp.float32),
    ],
)(_prop_body)


_BLK = 5120
_GRID = N_PAD // _BLK

_row_spec = pl.BlockSpec((_BLK, D), lambda i: (i, 0))
_col_spec = pl.BlockSpec((_BLK, 1), lambda i: (i, 0))
_w_spec = pl.BlockSpec((D, D), lambda i: (0, 0))
_b_spec = pl.BlockSpec((1, D), lambda i: (0, 0))


def _mm1_body(p0_ref, p1_ref, x_ref, w_ref, dinv_ref, y_ref):
    deg = p0_ref[...] + p1_ref[...] + 1.0
    dinv = lax.rsqrt(deg)
    dinv_ref[...] = dinv
    y_ref[...] = jnp.dot(x_ref[...], w_ref[...],
                         preferred_element_type=jnp.float32) * dinv


def _mm1_call(p0, p1, x, w):
    return pl.pallas_call(
        _mm1_body,
        grid=(_GRID,),
        in_specs=[_col_spec, _col_spec, _row_spec, _w_spec],
        out_specs=[_col_spec, _row_spec],
        out_shape=[
            jax.ShapeDtypeStruct((N_PAD, 1), jnp.float32),
            jax.ShapeDtypeStruct((N_PAD, D), jnp.float32),
        ],
    )(p0, p1, x, w)


def _mm2_body(a0_ref, a1_ref, y1_ref, dinv_ref, b_ref, w_ref, y2_ref):
    dinv = dinv_ref[...]
    h = (a0_ref[...] + a1_ref[...] + y1_ref[...]) * dinv + b_ref[...]
    h = jnp.maximum(h, 0.0)
    y2_ref[...] = jnp.dot(h, w_ref[...],
                          preferred_element_type=jnp.float32) * dinv


def _mm2_call(a0, a1, y1, dinv, b, w):
    return pl.pallas_call(
        _mm2_body,
        grid=(_GRID,),
        in_specs=[_row_spec, _row_spec, _row_spec, _col_spec, _b_spec, _w_spec],
        out_specs=_row_spec,
        out_shape=jax.ShapeDtypeStruct((N_PAD, D), jnp.float32),
    )(a0, a1, y1, dinv, b, w)


def _fin_body(a0_ref, a1_ref, y2_ref, dinv_ref, b_ref, out_ref):
    out_ref[...] = ((a0_ref[...] + a1_ref[...] + y2_ref[...]) * dinv_ref[...]
                    + b_ref[...])


_FBLK = 1000
_fin_row_spec = pl.BlockSpec((_FBLK, D), lambda i: (i, 0))
_fin_col_spec = pl.BlockSpec((_FBLK, 1), lambda i: (i, 0))


def _fin_call(a0, a1, y2, dinv, b):
    return pl.pallas_call(
        _fin_body,
        grid=(N_NODES // _FBLK,),
        in_specs=[_fin_row_spec, _fin_row_spec, _fin_row_spec,
                  _fin_col_spec, _b_spec],
        out_specs=_fin_row_spec,
        out_shape=jax.ShapeDtypeStruct((N_NODES, D), jnp.float32),
    )(a0, a1, y2, dinv, b)


@jax.jit
def _run(x, src2d, dst2d, W1, b1, W2, b2):
    b1r = b1.reshape(1, D)
    b2r = b2.reshape(1, D)

    degp = _deg_call(dst2d)
    p0 = degp[0].reshape(N_PAD, 1)
    p1 = degp[1].reshape(N_PAD, 1)
    dinv, y1 = _mm1_call(p0, p1, x, W1)

    acc1 = _prop_call(y1, src2d, dst2d)
    y2 = _mm2_call(acc1[0], acc1[1], y1, dinv, b1r, W2)

    acc2 = _prop_call(y2, src2d, dst2d)
    return _fin_call(acc2[0], acc2[1], y2, dinv, b2r)


def kernel(x, edge_index, W1, b1, W2, b2):
    src = edge_index[0].astype(jnp.int32)
    dst = edge_index[1].astype(jnp.int32)
    pad = N_NODES + (jnp.arange(E_PAD - N_EDGES, dtype=jnp.int32)
                     % (N_PAD - N_NODES))
    src2d = jnp.concatenate([src, pad]).reshape(-1, G)
    dst2d = jnp.concatenate([dst, pad]).reshape(-1, G)
    return _run(x, src2d, dst2d, W1, b1, W2, b2)


# speedup vs baseline: 1.0066x; 1.0053x over previous
import functools
import jax
import jax.numpy as jnp
from jax import lax
from jax.experimental import pallas as pl
from jax.experimental.pallas import tpu as pltpu
from jax.experimental.pallas import tpu_sc as plsc

N_NODES = 10000
N_PAD = 10240
N_EDGES = 320000
D = 128
NC = 2
NS = 16
NW = NC * NS
G = 128
GPT = 80
E_PAD = NW * GPT * G
NBUF = 2
ROWS_PER_TILE = N_PAD // NS

_MESH = plsc.VectorSubcoreMesh(core_axis_name="c", subcore_axis_name="s")


def _deg_body(dst_hbm, out_hbm, idx_v, ones_v, zvec_v, deg_sp, sem):
    c = lax.axis_index("c")
    s = lax.axis_index("s")
    wid = s * NC + c
    g0 = wid * GPT
    for j in range(G // 16):
        ones_v[pl.ds(j * 16, 16)] = jnp.ones((16,), jnp.float32)

    def zfill(i, carry):
        zvec_v[pl.ds(i * 16, 16)] = jnp.zeros((16,), jnp.float32)
        return carry

    lax.fori_loop(0, ROWS_PER_TILE // 16, zfill, 0)
    pltpu.sync_copy(dst_hbm.at[pl.ds(g0, GPT)], idx_v)
    pltpu.sync_copy(zvec_v, deg_sp.at[pl.ds(s * ROWS_PER_TILE, ROWS_PER_TILE)])
    plsc.subcore_barrier()

    def body(t, carry):
        for b in range(16):
            j = t * 16 + b
            pltpu.async_copy(ones_v, deg_sp.at[idx_v.at[j]], sem, add=True)
        for b in range(16):
            j = t * 16 + b
            pltpu.make_async_copy(ones_v, deg_sp.at[idx_v.at[j]], sem).wait()
        return carry

    lax.fori_loop(0, GPT // 16, body, 0)
    plsc.subcore_barrier()
    pltpu.sync_copy(deg_sp.at[pl.ds(s * ROWS_PER_TILE, ROWS_PER_TILE)],
                    out_hbm.at[c, pl.ds(s * ROWS_PER_TILE, ROWS_PER_TILE)])


_deg_call = functools.partial(
    pl.kernel,
    out_type=jax.ShapeDtypeStruct((NC, N_PAD), jnp.float32),
    mesh=_MESH,
    scratch_types=[
        pltpu.VMEM((GPT, G), jnp.int32),
        pltpu.VMEM((G,), jnp.float32),
        pltpu.VMEM((ROWS_PER_TILE,), jnp.float32),
        pltpu.VMEM_SHARED((N_PAD,), jnp.float32),
        pltpu.SemaphoreType.DMA,
    ],
)(_deg_body)


_HALF = GPT // 2


def _prop_body(y_hbm, src_hbm, dst_hbm, out_hbm,
               sidx_v, didx_v, msgs, gsems, acc_sp):
    c = lax.axis_index("c")
    s = lax.axis_index("s")
    wid = s * NC + c
    g0 = wid * GPT

    def zfill(i, carry):
        for k in range(D // 16):
            msgs[0][i, pl.ds(k * 16, 16)] = jnp.zeros((16,), jnp.float32)
        return carry

    lax.fori_loop(0, G, zfill, 0)
    for r in range(ROWS_PER_TILE // G):
        pltpu.sync_copy(msgs[0],
                        acc_sp.at[pl.ds(s * ROWS_PER_TILE + r * G, G)])
    plsc.subcore_barrier()

    def start_gather(j, b):
        pltpu.async_copy(y_hbm.at[sidx_v.at[j]], msgs[b], gsems[b])

    def wait_gather(j, b):
        pltpu.make_async_copy(y_hbm.at[sidx_v.at[j]], msgs[b], gsems[b]).wait()

    pltpu.sync_copy(src_hbm.at[pl.ds(g0, GPT)], sidx_v)
    for h in range(2):
        pltpu.sync_copy(dst_hbm.at[pl.ds(g0 + h * _HALF, _HALF)], didx_v)
        for b in range(NBUF):
            start_gather(h * _HALF + b, b)

        def body(t, carry):
            for b in range(NBUF):
                j = t * NBUF + b
                wait_gather(h * _HALF + j, b)
                pltpu.sync_copy(msgs[b], acc_sp.at[didx_v.at[j]], add=True)

                @pl.when(j + NBUF < _HALF)
                def _():
                    start_gather(h * _HALF + j + NBUF, b)
            return carry

        lax.fori_loop(0, _HALF // NBUF, body, 0)

    plsc.subcore_barrier()
    pltpu.sync_copy(acc_sp.at[pl.ds(s * ROWS_PER_TILE, ROWS_PER_TILE)],
                    out_hbm.at[c, pl.ds(s * ROWS_PER_TILE, ROWS_PER_TILE)])


_prop_call = functools.partial(
    pl.kernel,
    out_type=jax.ShapeDtypeStruct((NC, N_PAD, D), jnp.float32),
    mesh=_MESH,
    scratch_types=[
        pltpu.VMEM((GPT, G), jnp.int32),
        pltpu.VMEM((_HALF, G), jnp.int32),
        [pltpu.VMEM((G, D), jnp.float32) for _ in range(NBUF)],
        [pltpu.SemaphoreType.DMA for _ in range(NBUF)],
        pltpu.VMEM_SHARED((N_PAD, D), jnp.float32),
    ],
)(_prop_body)


_BLK = 5120
_GRID = N_PAD // _BLK

_row_spec = pl.BlockSpec((_BLK, D), lambda i: (i, 0))
_col_spec = pl.BlockSpec((_BLK, 1), lambda i: (i, 0))
_w_spec = pl.BlockSpec((D, D), lambda i: (0, 0))
_b_spec = pl.BlockSpec((1, D), lambda i: (0, 0))


def _mm1_body(p0_ref, p1_ref, x_ref, w_ref, dinv_ref, y_ref):
    deg = p0_ref[...] + p1_ref[...] + 1.0
    dinv = lax.rsqrt(deg)
    dinv_ref[...] = dinv
    y_ref[...] = jnp.dot(x_ref[...], w_ref[...],
                         preferred_element_type=jnp.float32) * dinv


def _mm1_call(p0, p1, x, w):
    return pl.pallas_call(
        _mm1_body,
        grid=(_GRID,),
        in_specs=[_col_spec, _col_spec, _row_spec, _w_spec],
        out_specs=[_col_spec, _row_spec],
        out_shape=[
            jax.ShapeDtypeStruct((N_PAD, 1), jnp.float32),
            jax.ShapeDtypeStruct((N_PAD, D), jnp.float32),
        ],
    )(p0, p1, x, w)


def _mm2_body(a0_ref, a1_ref, y1_ref, dinv_ref, b_ref, w_ref, y2_ref):
    dinv = dinv_ref[...]
    h = (a0_ref[...] + a1_ref[...] + y1_ref[...]) * dinv + b_ref[...]
    h = jnp.maximum(h, 0.0)
    y2_ref[...] = jnp.dot(h, w_ref[...],
                          preferred_element_type=jnp.float32) * dinv


def _mm2_call(a0, a1, y1, dinv, b, w):
    return pl.pallas_call(
        _mm2_body,
        grid=(_GRID,),
        in_specs=[_row_spec, _row_spec, _row_spec, _col_spec, _b_spec, _w_spec],
        out_specs=_row_spec,
        out_shape=jax.ShapeDtypeStruct((N_PAD, D), jnp.float32),
    )(a0, a1, y1, dinv, b, w)


def _fin_body(a0_ref, a1_ref, y2_ref, dinv_ref, b_ref, out_ref):
    out_ref[...] = ((a0_ref[...] + a1_ref[...] + y2_ref[...]) * dinv_ref[...]
                    + b_ref[...])


_FBLK = 1000
_fin_row_spec = pl.BlockSpec((_FBLK, D), lambda i: (i, 0))
_fin_col_spec = pl.BlockSpec((_FBLK, 1), lambda i: (i, 0))


def _fin_call(a0, a1, y2, dinv, b):
    return pl.pallas_call(
        _fin_body,
        grid=(N_NODES // _FBLK,),
        in_specs=[_fin_row_spec, _fin_row_spec, _fin_row_spec,
                  _fin_col_spec, _b_spec],
        out_specs=_fin_row_spec,
        out_shape=jax.ShapeDtypeStruct((N_NODES, D), jnp.float32),
    )(a0, a1, y2, dinv, b)


@jax.jit
def _run(x, src2d, dst2d, W1, b1, W2, b2):
    b1r = b1.reshape(1, D)
    b2r = b2.reshape(1, D)

    degp = _deg_call(dst2d)
    p0 = degp[0].reshape(N_PAD, 1)
    p1 = degp[1].reshape(N_PAD, 1)
    dinv, y1 = _mm1_call(p0, p1, x, W1)

    acc1 = _prop_call(y1, src2d, dst2d)
    y2 = _mm2_call(acc1[0], acc1[1], y1, dinv, b1r, W2)

    acc2 = _prop_call(y2, src2d, dst2d)
    return _fin_call(acc2[0], acc2[1], y2, dinv, b2r)


def kernel(x, edge_index, W1, b1, W2, b2):
    src = edge_index[0].astype(jnp.int32)
    dst = edge_index[1].astype(jnp.int32)
    pad = N_NODES + (jnp.arange(E_PAD - N_EDGES, dtype=jnp.int32)
                     % (N_PAD - N_NODES))
    src2d = jnp.concatenate([src, pad]).reshape(-1, G)
    dst2d = jnp.concatenate([dst, pad]).reshape(-1, G)
    return _run(x, src2d, dst2d, W1, b1, W2, b2)
